# baseline (speedup 1.0000x reference)
import functools

import numpy as np
import jax
import jax.numpy as jnp
from jax import lax
from jax.experimental import pallas as pl
from jax.experimental.pallas import tpu as pltpu
from jax.experimental.pallas import tpu_sc as plsc

_N = 10000
_E = 160000
_F_IN = 32
_F_EDGE = 16
_C_OUT = 16
_EDGE_H = 64
_D1 = 64
_D2 = 8
_G = 64

_NW = 32
_CH = 128
_CPW = 40
_EP = _NW * _CPW * _CH
_EB = 2048
_GRID = _EP // _EB

_HI = lax.Precision.HIGHEST

_R_np = np.zeros((_F_IN, _F_IN * _C_OUT), np.float32)
for _i in range(_F_IN):
    _R_np[_i, _i * _C_OUT:(_i + 1) * _C_OUT] = 1.0
_S_np = np.zeros((_F_IN * _C_OUT, _C_OUT), np.float32)
for _i in range(_F_IN):
    for _o in range(_C_OUT):
        _S_np[_i * _C_OUT + _o, _o] = 1.0


def _sc_gather(node_attr, batching, src2d, dst2d):
    mesh = plsc.VectorSubcoreMesh(core_axis_name="c", subcore_axis_name="s")

    @functools.partial(
        pl.kernel,
        mesh=mesh,
        compiler_params=pltpu.CompilerParams(use_tc_tiling_on_sc=False),
        out_type=[
            jax.ShapeDtypeStruct((_EP, _F_IN), jnp.float32),
            jax.ShapeDtypeStruct((_EP, 16), jnp.int32),
        ],
        scratch_types=[
            pltpu.VMEM((_CH,), jnp.int32),
            pltpu.VMEM((_CH, _F_IN), jnp.float32),
            pltpu.VMEM((_CH, 16), jnp.int32),
            pltpu.SemaphoreType.DMA,
        ],
    )
    def k(na_hbm, bat_hbm, src_hbm, dst_hbm, xj_hbm, gb_hbm,
          idx_v, rows_v, gbrows_v, sem):
        wid = lax.axis_index("s") * 2 + lax.axis_index("c")

        def body(c, carry):
            row = wid * _CPW + c
            pltpu.sync_copy(src_hbm.at[row], idx_v)
            pltpu.async_copy(na_hbm.at[idx_v], rows_v, sem).wait()
            pltpu.sync_copy(rows_v, xj_hbm.at[pl.ds(row * _CH, _CH)])
            pltpu.sync_copy(dst_hbm.at[row], idx_v)
            pltpu.async_copy(bat_hbm.at[idx_v], gbrows_v, sem).wait()
            pltpu.sync_copy(gbrows_v, gb_hbm.at[pl.ds(row * _CH, _CH)])
            return carry

        lax.fori_loop(0, _CPW, body, 0)

    return k(node_attr, batching, src2d, dst2d)


def _tc_body(ea_ref, xj_ref, gb_ref, na_ref, bat_ref,
             W1_ref, b1_ref, W2_ref, b2_ref, Wr_ref, bc_ref,
             Wd1_ref, bd1_ref, Wd2_ref, bd2_ref, R_ref, S_ref,
             out_ref, acc_ref):
    i = pl.program_id(0)

    @pl.when(i == 0)
    def _init():
        ohB = (bat_ref[...] == lax.broadcasted_iota(
            jnp.int32, (1, _G), 1)).astype(jnp.float32)
        t = lax.dot_general(ohB, na_ref[...],
                            (((0,), (0,)), ((), ())), precision=_HI)
        gnode = jnp.dot(t, Wr_ref[...], precision=_HI)
        cnt = lax.dot_general(ohB, jnp.ones((_N, 1), jnp.float32),
                              (((0,), (0,)), ((), ())), precision=_HI)
        acc_ref[...] = gnode + cnt * bc_ref[...]

    h = jnp.maximum(
        jnp.dot(ea_ref[...], W1_ref[...], precision=_HI) + b1_ref[...], 0.0)
    We = jnp.dot(h, W2_ref[...], precision=_HI) + b2_ref[...]
    xr = jnp.dot(xj_ref[...], R_ref[...], precision=_HI)
    msg = jnp.dot(xr * We, S_ref[...], precision=_HI)

    eid = i * _EB + lax.broadcasted_iota(jnp.int32, (_EB, 1), 0)
    valid = eid < _E
    oh = ((gb_ref[:, 0:1] == lax.broadcasted_iota(jnp.int32, (1, _G), 1))
          & valid).astype(jnp.float32)
    acc_ref[...] += lax.dot_general(oh, msg,
                                    (((0,), (0,)), ((), ())), precision=_HI)

    @pl.when(i == _GRID - 1)
    def _fin():
        g = acc_ref[...]
        gr = jnp.maximum(
            jnp.dot(g, Wd1_ref[...], precision=_HI) + bd1_ref[...], 0.0)
        out_ref[...] = jnp.dot(gr, Wd2_ref[...], precision=_HI) + bd2_ref[...]


def _tc_main(ea_p, xj, gb2, node_attr, bat2,
             W1, b1, W2, b2, W_root, b_conv, Wd1, bd1, Wd2, bd2, R, S):
    full = lambda shape: pl.BlockSpec(shape, lambda i: (0,) * len(shape))
    return pl.pallas_call(
        _tc_body,
        grid=(_GRID,),
        in_specs=[
            pl.BlockSpec((_EB, _F_EDGE), lambda i: (i, 0)),
            pl.BlockSpec((_EB, _F_IN), lambda i: (i, 0)),
            pl.BlockSpec((_EB, 16), lambda i: (i, 0)),
            full((_N, _F_IN)),
            full((_N, 1)),
            full((_F_EDGE, _EDGE_H)),
            full((1, _EDGE_H)),
            full((_EDGE_H, _F_IN * _C_OUT)),
            full((1, _F_IN * _C_OUT)),
            full((_F_IN, _C_OUT)),
            full((1, _C_OUT)),
            full((_C_OUT, _D1)),
            full((1, _D1)),
            full((_D1, _D2)),
            full((1, _D2)),
            full((_F_IN, _F_IN * _C_OUT)),
            full((_F_IN * _C_OUT, _C_OUT)),
        ],
        out_specs=full((_G, _D2)),
        out_shape=jax.ShapeDtypeStruct((_G, _D2), jnp.float32),
        scratch_shapes=[pltpu.VMEM((_G, _C_OUT), jnp.float32)],
    )(ea_p, xj, gb2, node_attr, bat2,
      W1, b1, W2, b2, W_root, b_conv, Wd1, bd1, Wd2, bd2, R, S)


def kernel(node_attr, edge_index, edge_attr, batching, W1, b1, W2, b2,
           W_root, b_conv, Wd1, bd1, Wd2, bd2):
    pad = _EP - _E
    src2d = jnp.concatenate(
        [edge_index[0], jnp.zeros((pad,), jnp.int32)]).reshape(_NW * _CPW, _CH)
    dst2d = jnp.concatenate(
        [edge_index[1], jnp.zeros((pad,), jnp.int32)]).reshape(_NW * _CPW, _CH)
    ea_p = jnp.concatenate(
        [edge_attr, jnp.zeros((pad, _F_EDGE), jnp.float32)])

    bat16 = jnp.broadcast_to(batching[:, None], (_N, 16))
    xj, gb = _sc_gather(node_attr, bat16, src2d, dst2d)

    R = jnp.asarray(_R_np)
    S = jnp.asarray(_S_np)
    return _tc_main(
        ea_p, xj, gb, node_attr, batching.reshape(_N, 1),
        W1, b1.reshape(1, -1), W2, b2.reshape(1, -1),
        W_root, b_conv.reshape(1, -1), Wd1, bd1.reshape(1, -1),
        Wd2, bd2.reshape(1, -1), R, S)


# speedup vs baseline: 2.5641x; 2.5641x over previous
import functools

import numpy as np
import jax
import jax.numpy as jnp
from jax import lax
from jax.experimental import pallas as pl
from jax.experimental.pallas import tpu as pltpu
from jax.experimental.pallas import tpu_sc as plsc

_N = 10000
_E = 160000
_F_IN = 32
_F_EDGE = 16
_C_OUT = 16
_EDGE_H = 64
_D1 = 64
_D2 = 8
_G = 64

_NW = 32
_CH = 128
_CPW = 40
_EP = _NW * _CPW * _CH
_EB = 2048
_GRID = _EP // _EB

_DP = lax.Precision.DEFAULT

_R_np = np.zeros((_F_IN, _F_IN * _C_OUT), np.float32)
for _i in range(_F_IN):
    _R_np[_i, _i * _C_OUT:(_i + 1) * _C_OUT] = 1.0
_S_np = np.zeros((_F_IN * _C_OUT, _C_OUT), np.float32)
for _i in range(_F_IN):
    for _o in range(_C_OUT):
        _S_np[_i * _C_OUT + _o, _o] = 1.0


def _sc_gather(node_attr, batching, src2d, dst2d):
    mesh = plsc.VectorSubcoreMesh(core_axis_name="c", subcore_axis_name="s")

    @functools.partial(
        pl.kernel,
        mesh=mesh,
        compiler_params=pltpu.CompilerParams(use_tc_tiling_on_sc=False),
        out_type=[
            jax.ShapeDtypeStruct((_EP, _F_IN), jnp.float32),
            jax.ShapeDtypeStruct((_EP, 16), jnp.int32),
        ],
        scratch_types=[
            pltpu.VMEM((_CH,), jnp.int32),
            pltpu.VMEM((_CH, _F_IN), jnp.float32),
            pltpu.VMEM((_CH, 16), jnp.int32),
            pltpu.SemaphoreType.DMA,
        ],
    )
    def k(na_hbm, bat_hbm, src_hbm, dst_hbm, xj_hbm, gb_hbm,
          idx_v, rows_v, gbrows_v, sem):
        wid = lax.axis_index("s") * 2 + lax.axis_index("c")

        def body(c, carry):
            row = wid * _CPW + c
            pltpu.sync_copy(src_hbm.at[row], idx_v)
            pltpu.async_copy(na_hbm.at[idx_v], rows_v, sem).wait()
            pltpu.sync_copy(rows_v, xj_hbm.at[pl.ds(row * _CH, _CH)])
            pltpu.sync_copy(dst_hbm.at[row], idx_v)
            pltpu.async_copy(bat_hbm.at[idx_v], gbrows_v, sem).wait()
            pltpu.sync_copy(gbrows_v, gb_hbm.at[pl.ds(row * _CH, _CH)])
            return carry

        lax.fori_loop(0, _CPW, body, 0)

    return k(node_attr, batching, src2d, dst2d)


def _tc_body(ea_ref, xj_ref, gb_ref, na_ref, bat_ref,
             W1_ref, b1_ref, W2_ref, b2_ref, Wr_ref, bc_ref,
             Wd1_ref, bd1_ref, Wd2_ref, bd2_ref, R_ref, S_ref,
             out_ref, acc_ref):
    i = pl.program_id(0)

    @pl.when(i == 0)
    def _init():
        ohB = (bat_ref[...] == lax.broadcasted_iota(
            jnp.int32, (1, _G), 1)).astype(jnp.float32)
        t = lax.dot_general(ohB, na_ref[...],
                            (((0,), (0,)), ((), ())), precision=_DP)
        gnode = jnp.dot(t, Wr_ref[...], precision=_DP)
        cnt = lax.dot_general(ohB, jnp.ones((_N, 1), jnp.float32),
                              (((0,), (0,)), ((), ())), precision=_DP)
        acc_ref[...] = gnode + cnt * bc_ref[...]

    h = jnp.maximum(
        jnp.dot(ea_ref[...], W1_ref[...], precision=_DP) + b1_ref[...], 0.0)
    We = jnp.dot(h, W2_ref[...], precision=_DP) + b2_ref[...]
    xr = jnp.dot(xj_ref[...], R_ref[...], precision=_DP)
    msg = jnp.dot(xr * We, S_ref[...], precision=_DP)

    eid = i * _EB + lax.broadcasted_iota(jnp.int32, (_EB, 1), 0)
    valid = eid < _E
    oh = ((gb_ref[:, 0:1] == lax.broadcasted_iota(jnp.int32, (1, _G), 1))
          & valid).astype(jnp.float32)
    acc_ref[...] += lax.dot_general(oh, msg,
                                    (((0,), (0,)), ((), ())), precision=_DP)

    @pl.when(i == _GRID - 1)
    def _fin():
        g = acc_ref[...]
        gr = jnp.maximum(
            jnp.dot(g, Wd1_ref[...], precision=_DP) + bd1_ref[...], 0.0)
        out_ref[...] = jnp.dot(gr, Wd2_ref[...], precision=_DP) + bd2_ref[...]


def _tc_main(ea_p, xj, gb2, node_attr, bat2,
             W1, b1, W2, b2, W_root, b_conv, Wd1, bd1, Wd2, bd2, R, S):
    full = lambda shape: pl.BlockSpec(shape, lambda i: (0,) * len(shape))
    return pl.pallas_call(
        _tc_body,
        grid=(_GRID,),
        in_specs=[
            pl.BlockSpec((_EB, _F_EDGE), lambda i: (i, 0)),
            pl.BlockSpec((_EB, _F_IN), lambda i: (i, 0)),
            pl.BlockSpec((_EB, 16), lambda i: (i, 0)),
            full((_N, _F_IN)),
            full((_N, 1)),
            full((_F_EDGE, _EDGE_H)),
            full((1, _EDGE_H)),
            full((_EDGE_H, _F_IN * _C_OUT)),
            full((1, _F_IN * _C_OUT)),
            full((_F_IN, _C_OUT)),
            full((1, _C_OUT)),
            full((_C_OUT, _D1)),
            full((1, _D1)),
            full((_D1, _D2)),
            full((1, _D2)),
            full((_F_IN, _F_IN * _C_OUT)),
            full((_F_IN * _C_OUT, _C_OUT)),
        ],
        out_specs=full((_G, _D2)),
        out_shape=jax.ShapeDtypeStruct((_G, _D2), jnp.float32),
        scratch_shapes=[pltpu.VMEM((_G, _C_OUT), jnp.float32)],
    )(ea_p, xj, gb2, node_attr, bat2,
      W1, b1, W2, b2, W_root, b_conv, Wd1, bd1, Wd2, bd2, R, S)


def kernel(node_attr, edge_index, edge_attr, batching, W1, b1, W2, b2,
           W_root, b_conv, Wd1, bd1, Wd2, bd2):
    pad = _EP - _E
    src2d = jnp.concatenate(
        [edge_index[0], jnp.zeros((pad,), jnp.int32)]).reshape(_NW * _CPW, _CH)
    dst2d = jnp.concatenate(
        [edge_index[1], jnp.zeros((pad,), jnp.int32)]).reshape(_NW * _CPW, _CH)
    ea_p = jnp.concatenate(
        [edge_attr, jnp.zeros((pad, _F_EDGE), jnp.float32)])

    bat16 = jnp.broadcast_to(batching[:, None], (_N, 16))
    xj, gb = _sc_gather(node_attr, bat16, src2d, dst2d)

    R = jnp.asarray(_R_np)
    S = jnp.asarray(_S_np)
    return _tc_main(
        ea_p, xj, gb, node_attr, batching.reshape(_N, 1),
        W1, b1.reshape(1, -1), W2, b2.reshape(1, -1),
        W_root, b_conv.reshape(1, -1), Wd1, bd1.reshape(1, -1),
        Wd2, bd2.reshape(1, -1), R, S)
